# baseline (speedup 1.0000x reference)
import functools

import jax
import jax.numpy as jnp
from jax.experimental import pallas as pl
from jax.experimental.pallas import tpu as pltpu

_SPLITS = 2
_SUB_ROWS = 256


def _layer_kernel(splits, sub_rows, *refs):
    a_refs = refs[:2 * splits]
    x_ref, nrm_ref, watt_ref, batt_ref, w_ref, b_ref, o_ref = refs[2 * splits:]
    x = x_ref[...].astype(jnp.bfloat16)
    watt = watt_ref[...]
    batt = batt_ref[0, 0]
    w = w_ref[...]
    b = b_ref[...]
    nrm_all = nrm_ref[...]
    for h in range(splits):
        lo = h * sub_rows
        t0 = jnp.dot(a_refs[h][0].astype(jnp.bfloat16), x,
                     preferred_element_type=jnp.float32)
        t1 = jnp.dot(a_refs[splits + h][0].astype(jnp.bfloat16), x,
                     preferred_element_type=jnp.float32)
        nrm = nrm_all[lo:lo + sub_rows]
        t0 = t0 / nrm[:, 0:1]
        t1 = t1 / nrm[:, 1:2]
        l0 = jnp.dot(t0, watt, preferred_element_type=jnp.float32) + batt
        l1 = jnp.dot(t1, watt, preferred_element_type=jnp.float32) + batt
        mx = jnp.maximum(l0, l1)
        e0 = jnp.exp(l0 - mx)
        e1 = jnp.exp(l1 - mx)
        comb = (t0 * e0 + t1 * e1) / (e0 + e1)
        out = jnp.dot(comb, w, preferred_element_type=jnp.float32) + b
        o_ref[lo:lo + sub_rows, :] = jnp.maximum(out, 0.0)


def _layer(x, motifs_all, nrm_t, w_att, b_att, w, b, *, interpret=False):
    n = x.shape[0]
    d_in = x.shape[1]
    d_out = w.shape[1]
    m = nrm_t.shape[1]
    splits, sub_rows = _SPLITS, _SUB_ROWS
    block_rows = splits * sub_rows
    grid = (n // block_rows,)
    a_specs = [
        pl.BlockSpec((1, sub_rows, n),
                     lambda i, mm=mm, h=h: (mm, splits * i + h, 0))
        for mm in range(2) for h in range(splits)
    ]
    return pl.pallas_call(
        functools.partial(_layer_kernel, splits, sub_rows),
        grid=grid,
        in_specs=a_specs + [
            pl.BlockSpec((n, d_in), lambda i: (0, 0)),
            pl.BlockSpec((block_rows, m), lambda i: (i, 0)),
            pl.BlockSpec((d_in, 1), lambda i: (0, 0)),
            pl.BlockSpec((1, 1), lambda i: (0, 0)),
            pl.BlockSpec((d_in, d_out), lambda i: (0, 0)),
            pl.BlockSpec((1, d_out), lambda i: (0, 0)),
        ],
        out_specs=pl.BlockSpec((block_rows, d_out), lambda i: (i, 0)),
        out_shape=jax.ShapeDtypeStruct((n, d_out), jnp.float32),
        compiler_params=pltpu.CompilerParams(
            dimension_semantics=("arbitrary",)),
        interpret=interpret,
    )(*([motifs_all] * (2 * splits)), x, nrm_t, w_att, b_att, w, b)


@jax.jit
def kernel(x, motifs_all, motifs_num, w_att0, b_att0, W0, b0,
           w_att1, b_att1, W1, b1):
    nrm_t = motifs_num.T
    b_att0 = b_att0.reshape(1, 1)
    b_att1 = b_att1.reshape(1, 1)
    b0 = b0.reshape(1, -1)
    b1 = b1.reshape(1, -1)
    h = _layer(x, motifs_all, nrm_t, w_att0, b_att0, W0, b0)
    return _layer(h, motifs_all, nrm_t, w_att1, b_att1, W1, b1)
